# baseline (speedup 1.0000x reference)
import functools

import jax
import jax.numpy as jnp
from jax import lax
from jax.experimental import pallas as pl
from jax.experimental.pallas import tpu as pltpu
from jax.experimental.pallas import tpu_sc as plsc

N = 10000
E = 320000
D = 128
NGRAPH = 64
NSC = 2
NTILE = 16
NW = NSC * NTILE
EPT = E // NW
CHUNK = 80
NCHUNK = EPT // CHUNK
NPAD = 10240
RPT = NPAD // NTILE

_mesh = plsc.VectorSubcoreMesh(core_axis_name="c", subcore_axis_name="s")


@functools.partial(
    pl.kernel,
    mesh=_mesh,
    out_type=jax.ShapeDtypeStruct((NSC, NPAD, D), jnp.float32),
    scratch_types=[
        pltpu.VMEM((NCHUNK, CHUNK), jnp.int32),
        pltpu.VMEM((CHUNK, D), jnp.float32),
        pltpu.VMEM_SHARED((NPAD, D), jnp.float32),
    ],
)
def _deg_sc(dst_hbm, ones_hbm, zeros_hbm, out_hbm, dbuf, onesbuf, acc):
    cid = lax.axis_index("c")
    sid = lax.axis_index("s")
    wid = sid * NSC + cid
    pltpu.sync_copy(dst_hbm.at[wid], dbuf)
    pltpu.sync_copy(ones_hbm, onesbuf)
    pltpu.sync_copy(zeros_hbm, acc.at[pl.ds(sid * RPT, RPT)])
    plsc.subcore_barrier()

    def body(i, carry):
        pltpu.sync_copy(onesbuf, acc.at[dbuf.at[i]], add=True)
        return carry

    lax.fori_loop(0, NCHUNK, body, 0)
    plsc.subcore_barrier()
    pltpu.sync_copy(acc.at[pl.ds(sid * RPT, RPT)],
                    out_hbm.at[cid, pl.ds(sid * RPT, RPT)])


@functools.partial(
    pl.kernel,
    mesh=_mesh,
    out_type=jax.ShapeDtypeStruct((NSC, NPAD, D), jnp.float32),
    scratch_types=[
        pltpu.VMEM((EPT,), jnp.int32),
        pltpu.VMEM((NCHUNK, CHUNK), jnp.int32),
        pltpu.VMEM((CHUNK, D), jnp.float32),
        pltpu.VMEM((CHUNK, D), jnp.float32),
        pltpu.VMEM_SHARED((NPAD, D), jnp.float32),
        pltpu.SemaphoreType.DMA,
        pltpu.SemaphoreType.DMA,
        pltpu.SemaphoreType.DMA,
        pltpu.SemaphoreType.DMA,
    ],
)
def _agg_sc(g_hbm, src_hbm, dst_hbm, zeros_hbm, out_hbm, sbuf, dbuf, rows0,
            rows1, acc, gsem0, gsem1, ssem0, ssem1):
    cid = lax.axis_index("c")
    sid = lax.axis_index("s")
    wid = sid * NSC + cid
    pltpu.sync_copy(src_hbm.at[pl.ds(wid * EPT, EPT)], sbuf)
    pltpu.sync_copy(dst_hbm.at[wid], dbuf)
    pltpu.sync_copy(zeros_hbm, acc.at[pl.ds(sid * RPT, RPT)])
    plsc.subcore_barrier()

    bufs = (rows0, rows1)
    gsems = (gsem0, gsem1)
    ssems = (ssem0, ssem1)

    def _gather(c, b):
        pltpu.async_copy(
            g_hbm.at[sbuf.at[pl.ds(c * CHUNK, CHUNK)]], bufs[b], gsems[b])

    def _gather_wait(c, b):
        pltpu.make_async_copy(
            g_hbm.at[sbuf.at[pl.ds(c * CHUNK, CHUNK)]], bufs[b],
            gsems[b]).wait()

    def _scatter(c, b):
        pltpu.async_copy(bufs[b], acc.at[dbuf.at[c]], ssems[b], add=True)

    def _scatter_wait(c, b):
        pltpu.make_async_copy(bufs[b], acc.at[dbuf.at[c]], ssems[b]).wait()

    _gather(0, 0)
    _gather(1, 1)
    _gather_wait(0, 0)
    _scatter(0, 0)

    def body(j, carry):
        for k in (0, 1):
            c = 1 + 2 * j + k
            b = (1 + k) % 2
            _gather_wait(c, b)
            _scatter(c, b)
            _scatter_wait(c - 1, 1 - b)

            @pl.when(c + 1 < NCHUNK)
            def _():
                _gather(c + 1, 1 - b)
        return carry

    lax.fori_loop(0, (NCHUNK - 1) // 2, body, 0)
    _scatter_wait(NCHUNK - 1, (NCHUNK - 1) % 2)
    plsc.subcore_barrier()
    pltpu.sync_copy(acc.at[pl.ds(sid * RPT, RPT)],
                    out_hbm.at[cid, pl.ds(sid * RPT, RPT)])


_BN = 1000
_BNP = 2000


def _dinv_block(dpv):
    deg = dpv[0, :, 0:1] + dpv[1, :, 0:1] + 1.0
    return lax.rsqrt(deg)


def _mm_body(x_ref, w_ref, o_ref):
    o_ref[...] = jnp.dot(x_ref[...], w_ref[...],
                         preferred_element_type=jnp.float32)


def _scale_body(h_ref, dp_ref, o_ref, dinv_ref):
    dinv = _dinv_block(dp_ref[...])
    dinv_ref[...] = jnp.broadcast_to(dinv, dinv_ref.shape)
    o_ref[...] = h_ref[...] * dinv


def _layer_body(p_ref, g_ref, dinv_ref, b_ref, w_ref, o_ref):
    dinv = dinv_ref[...]
    pv = p_ref[...]
    a = jnp.maximum((pv[0] + pv[1] + g_ref[...]) * dinv + b_ref[...], 0.0)
    o_ref[...] = jnp.dot(a, w_ref[...],
                         preferred_element_type=jnp.float32) * dinv


def _final_body(p_ref, g_ref, dinv_ref, b_ref, batch_ref, wm1_ref, bm1_ref,
                wm2_ref, bm2_ref, o_ref, sums, cnt):
    i = pl.program_id(0)

    @pl.when(i == 0)
    def _():
        sums[...] = jnp.zeros_like(sums)
        cnt[...] = jnp.zeros_like(cnt)

    dinv = dinv_ref[...]
    pv = p_ref[...]
    a = jnp.maximum((pv[0] + pv[1] + g_ref[...]) * dinv + b_ref[...], 0.0)
    gid = lax.broadcasted_iota(jnp.int32, (_BNP, NGRAPH), 1)
    oh = (batch_ref[...] == gid).astype(jnp.float32)
    sums[...] += lax.dot_general(oh, a, (((0,), (0,)), ((), ())),
                                 preferred_element_type=jnp.float32)
    cnt[...] += jnp.broadcast_to(jnp.sum(oh, axis=0)[:, None], (NGRAPH, D))

    @pl.when(i == pl.num_programs(0) - 1)
    def _():
        pooled = sums[...] / jnp.maximum(cnt[...], 1.0)
        z = jnp.maximum(jnp.dot(pooled, wm1_ref[...],
                                preferred_element_type=jnp.float32)
                        + bm1_ref[...], 0.0)
        o_ref[...] = jnp.dot(z, wm2_ref[...],
                             preferred_element_type=jnp.float32) + bm2_ref[...]


def _mm(x, w):
    return pl.pallas_call(
        _mm_body,
        grid=(N // _BN,),
        in_specs=[pl.BlockSpec((_BN, D), lambda i: (i, 0)),
                  pl.BlockSpec((D, D), lambda i: (0, 0))],
        out_specs=pl.BlockSpec((_BN, D), lambda i: (i, 0)),
        out_shape=jax.ShapeDtypeStruct((N, D), jnp.float32),
    )(x, w)


def _scale(h, degp):
    return pl.pallas_call(
        _scale_body,
        grid=(N // _BN,),
        in_specs=[pl.BlockSpec((_BN, D), lambda i: (i, 0)),
                  pl.BlockSpec((NSC, _BN, D), lambda i: (0, i, 0))],
        out_specs=[pl.BlockSpec((_BN, D), lambda i: (i, 0)),
                   pl.BlockSpec((_BN, D), lambda i: (i, 0))],
        out_shape=[jax.ShapeDtypeStruct((N, D), jnp.float32),
                   jax.ShapeDtypeStruct((N, D), jnp.float32)],
    )(h, degp)


def _layer(p, g, dinvb, b, w):
    return pl.pallas_call(
        _layer_body,
        grid=(N // _BN,),
        in_specs=[pl.BlockSpec((NSC, _BN, D), lambda i: (0, i, 0)),
                  pl.BlockSpec((_BN, D), lambda i: (i, 0)),
                  pl.BlockSpec((_BN, D), lambda i: (i, 0)),
                  pl.BlockSpec((1, D), lambda i: (0, 0)),
                  pl.BlockSpec((D, D), lambda i: (0, 0))],
        out_specs=pl.BlockSpec((_BN, D), lambda i: (i, 0)),
        out_shape=jax.ShapeDtypeStruct((N, D), jnp.float32),
    )(p, g, dinvb, b, w)


def _final(p, g, dinvb, b, batch2, wm1, bm1, wm2, bm2):
    nhid = wm1.shape[1]
    return pl.pallas_call(
        _final_body,
        grid=(N // _BNP,),
        in_specs=[pl.BlockSpec((NSC, _BNP, D), lambda i: (0, i, 0)),
                  pl.BlockSpec((_BNP, D), lambda i: (i, 0)),
                  pl.BlockSpec((_BNP, D), lambda i: (i, 0)),
                  pl.BlockSpec((1, D), lambda i: (0, 0)),
                  pl.BlockSpec((_BNP, 1), lambda i: (i, 0)),
                  pl.BlockSpec((D, nhid), lambda i: (0, 0)),
                  pl.BlockSpec((1, nhid), lambda i: (0, 0)),
                  pl.BlockSpec((nhid, D), lambda i: (0, 0)),
                  pl.BlockSpec((1, D), lambda i: (0, 0))],
        out_specs=pl.BlockSpec((NGRAPH, D), lambda i: (0, 0)),
        out_shape=jax.ShapeDtypeStruct((NGRAPH, D), jnp.float32),
        scratch_shapes=[pltpu.VMEM((NGRAPH, D), jnp.float32),
                        pltpu.VMEM((NGRAPH, D), jnp.float32)],
    )(p, g, dinvb, b, batch2, wm1, bm1, wm2, bm2)


def kernel(x, edge_index, batch, W0, b0, Wh0, bh0, Wh1, bh1, Wh2, bh2,
           Wm1, bm1, Wm2, bm2):
    src_flat = edge_index[0]
    dst3 = edge_index[1].reshape(NW, NCHUNK, CHUNK)
    zerosd = jnp.zeros((RPT, D), jnp.float32)
    onesd = jnp.ones((CHUNK, D), jnp.float32)

    degp = _deg_sc(dst3, onesd, zerosd)
    h = _mm(x, W0)
    g, dinvb = _scale(h, degp)
    for b, w in ((b0, Wh0), (bh0, Wh1), (bh1, Wh2)):
        p = _agg_sc(g, src_flat, dst3, zerosd)
        g = _layer(p, g, dinvb, b.reshape(1, D), w)
    p = _agg_sc(g, src_flat, dst3, zerosd)
    return _final(p, g, dinvb, bh2.reshape(1, D), batch.reshape(N, 1),
                  Wm1, bm1.reshape(1, -1), Wm2, bm2.reshape(1, D))


# speedup vs baseline: 1.2357x; 1.2357x over previous
import functools

import jax
import jax.numpy as jnp
from jax import lax
from jax.experimental import pallas as pl
from jax.experimental.pallas import tpu as pltpu
from jax.experimental.pallas import tpu_sc as plsc

N = 10000
E = 320000
D = 128
NGRAPH = 64
NSC = 2
NTILE = 16
NW = NSC * NTILE
EPT = E // NW
CHUNK = 80
NCHUNK = EPT // CHUNK
NPAD = 10240
RPT = NPAD // NTILE

_mesh = plsc.VectorSubcoreMesh(core_axis_name="c", subcore_axis_name="s")


@functools.partial(
    pl.kernel,
    mesh=_mesh,
    out_type=jax.ShapeDtypeStruct((NSC, NPAD, D), jnp.float32),
    scratch_types=[
        pltpu.VMEM((NCHUNK, CHUNK), jnp.int32),
        pltpu.VMEM((CHUNK, D), jnp.float32),
        pltpu.VMEM_SHARED((NPAD, D), jnp.float32),
    ],
)
def _deg_sc(dst_hbm, ones_hbm, zeros_hbm, out_hbm, dbuf, onesbuf, acc):
    cid = lax.axis_index("c")
    sid = lax.axis_index("s")
    wid = sid * NSC + cid
    pltpu.sync_copy(dst_hbm.at[wid], dbuf)
    pltpu.sync_copy(ones_hbm, onesbuf)
    pltpu.sync_copy(zeros_hbm, acc.at[pl.ds(sid * RPT, RPT)])
    plsc.subcore_barrier()

    def body(i, carry):
        pltpu.sync_copy(onesbuf, acc.at[dbuf.at[i]], add=True)
        return carry

    lax.fori_loop(0, NCHUNK, body, 0)
    plsc.subcore_barrier()
    pltpu.sync_copy(acc.at[pl.ds(sid * RPT, RPT)],
                    out_hbm.at[cid, pl.ds(sid * RPT, RPT)])


@functools.partial(
    pl.kernel,
    mesh=_mesh,
    out_type=jax.ShapeDtypeStruct((NSC, NPAD, D), jnp.float32),
    scratch_types=[
        pltpu.VMEM((EPT,), jnp.int32),
        pltpu.VMEM((NCHUNK, CHUNK), jnp.int32),
        pltpu.VMEM((CHUNK, D), jnp.float32),
        pltpu.VMEM((CHUNK, D), jnp.float32),
        pltpu.VMEM_SHARED((NPAD, D), jnp.float32),
        pltpu.SemaphoreType.DMA,
        pltpu.SemaphoreType.DMA,
    ],
)
def _agg_sc(g_hbm, src_hbm, dst_hbm, zeros_hbm, out_hbm, sbuf, dbuf, rows0,
            rows1, acc, sem0, sem1):
    cid = lax.axis_index("c")
    sid = lax.axis_index("s")
    wid = sid * NSC + cid
    pltpu.sync_copy(src_hbm.at[pl.ds(wid * EPT, EPT)], sbuf)
    pltpu.sync_copy(dst_hbm.at[wid], dbuf)
    pltpu.sync_copy(zeros_hbm, acc.at[pl.ds(sid * RPT, RPT)])
    plsc.subcore_barrier()

    bufs = (rows0, rows1)
    sems = (sem0, sem1)

    def _gather(c, b):
        return pltpu.async_copy(
            g_hbm.at[sbuf.at[pl.ds(c * CHUNK, CHUNK)]], bufs[b], sems[b])

    _gather(0, 0)

    def body(j, carry):
        for b in (0, 1):
            c = 2 * j + b
            _gather(c + 1, 1 - b)
            pltpu.make_async_copy(
                g_hbm.at[sbuf.at[pl.ds(c * CHUNK, CHUNK)]], bufs[b],
                sems[b]).wait()
            pltpu.sync_copy(bufs[b], acc.at[dbuf.at[c]], add=True)
        return carry

    lax.fori_loop(0, NCHUNK // 2, body, 0)
    last = NCHUNK - 1
    pltpu.make_async_copy(
        g_hbm.at[sbuf.at[pl.ds(last * CHUNK, CHUNK)]], bufs[last % 2],
        sems[last % 2]).wait()
    pltpu.sync_copy(bufs[last % 2], acc.at[dbuf.at[last]], add=True)
    plsc.subcore_barrier()
    pltpu.sync_copy(acc.at[pl.ds(sid * RPT, RPT)],
                    out_hbm.at[cid, pl.ds(sid * RPT, RPT)])


_BN = 1000
_BNP = 2000


def _dinv_block(dpv):
    deg = dpv[0, :, 0:1] + dpv[1, :, 0:1] + 1.0
    return lax.rsqrt(deg)


def _mm_scale_body(x_ref, w_ref, dp_ref, o_ref, dinv_ref):
    dinv = _dinv_block(dp_ref[...])
    dinv_ref[...] = jnp.broadcast_to(dinv, dinv_ref.shape)
    o_ref[...] = jnp.dot(x_ref[...], w_ref[...],
                         preferred_element_type=jnp.float32) * dinv


def _layer_body(p_ref, g_ref, dinv_ref, b_ref, w_ref, o_ref):
    dinv = dinv_ref[...]
    pv = p_ref[...]
    a = jnp.maximum((pv[0] + pv[1] + g_ref[...]) * dinv + b_ref[...], 0.0)
    o_ref[...] = jnp.dot(a, w_ref[...],
                         preferred_element_type=jnp.float32) * dinv


def _final_body(p_ref, g_ref, dinv_ref, b_ref, batch_ref, wm1_ref, bm1_ref,
                wm2_ref, bm2_ref, o_ref, sums, cnt):
    i = pl.program_id(0)

    @pl.when(i == 0)
    def _():
        sums[...] = jnp.zeros_like(sums)
        cnt[...] = jnp.zeros_like(cnt)

    dinv = dinv_ref[...]
    pv = p_ref[...]
    a = jnp.maximum((pv[0] + pv[1] + g_ref[...]) * dinv + b_ref[...], 0.0)
    gid = lax.broadcasted_iota(jnp.int32, (_BNP, NGRAPH), 1)
    oh = (batch_ref[...] == gid).astype(jnp.float32)
    sums[...] += lax.dot_general(oh, a, (((0,), (0,)), ((), ())),
                                 preferred_element_type=jnp.float32)
    cnt[...] += jnp.broadcast_to(jnp.sum(oh, axis=0)[:, None], (NGRAPH, D))

    @pl.when(i == pl.num_programs(0) - 1)
    def _():
        pooled = sums[...] / jnp.maximum(cnt[...], 1.0)
        z = jnp.maximum(jnp.dot(pooled, wm1_ref[...],
                                preferred_element_type=jnp.float32)
                        + bm1_ref[...], 0.0)
        o_ref[...] = jnp.dot(z, wm2_ref[...],
                             preferred_element_type=jnp.float32) + bm2_ref[...]


def _mm_scale(x, w, degp):
    return pl.pallas_call(
        _mm_scale_body,
        grid=(N // _BN,),
        in_specs=[pl.BlockSpec((_BN, D), lambda i: (i, 0)),
                  pl.BlockSpec((D, D), lambda i: (0, 0)),
                  pl.BlockSpec((NSC, _BN, D), lambda i: (0, i, 0))],
        out_specs=[pl.BlockSpec((_BN, D), lambda i: (i, 0)),
                   pl.BlockSpec((_BN, D), lambda i: (i, 0))],
        out_shape=[jax.ShapeDtypeStruct((N, D), jnp.float32),
                   jax.ShapeDtypeStruct((N, D), jnp.float32)],
    )(x, w, degp)


def _layer(p, g, dinvb, b, w):
    return pl.pallas_call(
        _layer_body,
        grid=(N // _BN,),
        in_specs=[pl.BlockSpec((NSC, _BN, D), lambda i: (0, i, 0)),
                  pl.BlockSpec((_BN, D), lambda i: (i, 0)),
                  pl.BlockSpec((_BN, D), lambda i: (i, 0)),
                  pl.BlockSpec((1, D), lambda i: (0, 0)),
                  pl.BlockSpec((D, D), lambda i: (0, 0))],
        out_specs=pl.BlockSpec((_BN, D), lambda i: (i, 0)),
        out_shape=jax.ShapeDtypeStruct((N, D), jnp.float32),
    )(p, g, dinvb, b, w)


def _final(p, g, dinvb, b, batch2, wm1, bm1, wm2, bm2):
    nhid = wm1.shape[1]
    return pl.pallas_call(
        _final_body,
        grid=(N // _BNP,),
        in_specs=[pl.BlockSpec((NSC, _BNP, D), lambda i: (0, i, 0)),
                  pl.BlockSpec((_BNP, D), lambda i: (i, 0)),
                  pl.BlockSpec((_BNP, D), lambda i: (i, 0)),
                  pl.BlockSpec((1, D), lambda i: (0, 0)),
                  pl.BlockSpec((_BNP, 1), lambda i: (i, 0)),
                  pl.BlockSpec((D, nhid), lambda i: (0, 0)),
                  pl.BlockSpec((1, nhid), lambda i: (0, 0)),
                  pl.BlockSpec((nhid, D), lambda i: (0, 0)),
                  pl.BlockSpec((1, D), lambda i: (0, 0))],
        out_specs=pl.BlockSpec((NGRAPH, D), lambda i: (0, 0)),
        out_shape=jax.ShapeDtypeStruct((NGRAPH, D), jnp.float32),
        scratch_shapes=[pltpu.VMEM((NGRAPH, D), jnp.float32),
                        pltpu.VMEM((NGRAPH, D), jnp.float32)],
    )(p, g, dinvb, b, batch2, wm1, bm1, wm2, bm2)


def kernel(x, edge_index, batch, W0, b0, Wh0, bh0, Wh1, bh1, Wh2, bh2,
           Wm1, bm1, Wm2, bm2):
    src_flat = edge_index[0]
    dst3 = edge_index[1].reshape(NW, NCHUNK, CHUNK)
    zerosd = jnp.zeros((RPT, D), jnp.float32)
    onesd = jnp.ones((CHUNK, D), jnp.float32)

    degp = _deg_sc(dst3, onesd, zerosd)
    g, dinvb = _mm_scale(x, W0, degp)

    for b, w in ((b0, Wh0), (bh0, Wh1), (bh1, Wh2)):
        p = _agg_sc(g, src_flat, dst3, zerosd)
        g = _layer(p, g, dinvb, b.reshape(1, D), w)
    p = _agg_sc(g, src_flat, dst3, zerosd)
    return _final(p, g, dinvb, bh2.reshape(1, D), batch.reshape(N, 1),
                  Wm1, bm1.reshape(1, -1), Wm2, bm2.reshape(1, D))


# speedup vs baseline: 1.3325x; 1.0784x over previous
import functools

import jax
import jax.numpy as jnp
from jax import lax
from jax.experimental import pallas as pl
from jax.experimental.pallas import tpu as pltpu
from jax.experimental.pallas import tpu_sc as plsc

N = 10000
E = 320000
D = 128
NGRAPH = 64
NSC = 2
NTILE = 16
NW = NSC * NTILE
EPT = E // NW
CHUNK = 80
NCHUNK = EPT // CHUNK
NPAD = 10240
RPT = NPAD // NTILE

_mesh = plsc.VectorSubcoreMesh(core_axis_name="c", subcore_axis_name="s")


@functools.partial(
    pl.kernel,
    mesh=_mesh,
    out_type=jax.ShapeDtypeStruct((NSC * NPAD,), jnp.float32),
    scratch_types=[
        pltpu.VMEM((NCHUNK, CHUNK), jnp.int32),
        pltpu.VMEM((CHUNK,), jnp.float32),
        pltpu.VMEM((RPT,), jnp.float32),
        pltpu.VMEM_SHARED((NPAD,), jnp.float32),
    ],
)
def _deg_sc(dst_hbm, out_hbm, dbuf, onesbuf, zbuf, acc):
    cid = lax.axis_index("c")
    sid = lax.axis_index("s")
    wid = sid * NSC + cid
    pltpu.sync_copy(dst_hbm.at[wid], dbuf)

    def fill(i, carry):
        onesbuf[pl.ds(i * 16, 16)] = jnp.ones((16,), jnp.float32)
        return carry

    lax.fori_loop(0, CHUNK // 16, fill, 0)

    def zfill(i, carry):
        zbuf[pl.ds(i * 16, 16)] = jnp.zeros((16,), jnp.float32)
        return carry

    lax.fori_loop(0, RPT // 16, zfill, 0)
    pltpu.sync_copy(zbuf, acc.at[pl.ds(sid * RPT, RPT)])
    plsc.subcore_barrier()

    def body(i, carry):
        pltpu.sync_copy(onesbuf, acc.at[dbuf.at[i]], add=True)
        return carry

    lax.fori_loop(0, NCHUNK, body, 0)
    plsc.subcore_barrier()
    pltpu.sync_copy(acc.at[pl.ds(sid * RPT, RPT)],
                    out_hbm.at[pl.ds(cid * NPAD + sid * RPT, RPT)])


@functools.partial(
    pl.kernel,
    mesh=_mesh,
    out_type=jax.ShapeDtypeStruct((NSC, NPAD, D), jnp.float32),
    scratch_types=[
        pltpu.VMEM((EPT,), jnp.int32),
        pltpu.VMEM((NCHUNK, CHUNK), jnp.int32),
        pltpu.VMEM((CHUNK, D), jnp.float32),
        pltpu.VMEM((CHUNK, D), jnp.float32),
        pltpu.VMEM_SHARED((NPAD, D), jnp.float32),
        pltpu.SemaphoreType.DMA,
        pltpu.SemaphoreType.DMA,
    ],
)
def _agg_sc(g_hbm, src_hbm, dst_hbm, zeros_hbm, out_hbm, sbuf, dbuf, rows0,
            rows1, acc, sem0, sem1):
    cid = lax.axis_index("c")
    sid = lax.axis_index("s")
    wid = sid * NSC + cid
    pltpu.sync_copy(src_hbm.at[pl.ds(wid * EPT, EPT)], sbuf)
    pltpu.sync_copy(dst_hbm.at[wid], dbuf)
    pltpu.sync_copy(zeros_hbm, acc.at[pl.ds(sid * RPT, RPT)])
    plsc.subcore_barrier()

    bufs = (rows0, rows1)
    sems = (sem0, sem1)

    def _gather(c, b):
        return pltpu.async_copy(
            g_hbm.at[sbuf.at[pl.ds(c * CHUNK, CHUNK)]], bufs[b], sems[b])

    _gather(0, 0)

    def body(j, carry):
        for b in (0, 1):
            c = 2 * j + b
            _gather(c + 1, 1 - b)
            pltpu.make_async_copy(
                g_hbm.at[sbuf.at[pl.ds(c * CHUNK, CHUNK)]], bufs[b],
                sems[b]).wait()
            pltpu.sync_copy(bufs[b], acc.at[dbuf.at[c]], add=True)
        return carry

    lax.fori_loop(0, NCHUNK // 2, body, 0)
    last = NCHUNK - 1
    pltpu.make_async_copy(
        g_hbm.at[sbuf.at[pl.ds(last * CHUNK, CHUNK)]], bufs[last % 2],
        sems[last % 2]).wait()
    pltpu.sync_copy(bufs[last % 2], acc.at[dbuf.at[last]], add=True)
    plsc.subcore_barrier()
    pltpu.sync_copy(acc.at[pl.ds(sid * RPT, RPT)],
                    out_hbm.at[cid, pl.ds(sid * RPT, RPT)])


_BN = 1000
_BNP = 2000


def _dinv_block(dpv):
    deg = dpv[0, :, 0:1] + dpv[1, :, 0:1] + 1.0
    return lax.rsqrt(deg)


def _mm_scale_body(x_ref, w_ref, dp_ref, o_ref, dinv_ref):
    dinv = _dinv_block(dp_ref[...])
    dinv_ref[...] = jnp.broadcast_to(dinv, dinv_ref.shape)
    o_ref[...] = jnp.dot(x_ref[...], w_ref[...],
                         preferred_element_type=jnp.float32) * dinv


def _layer_body(p_ref, g_ref, dinv_ref, b_ref, w_ref, o_ref):
    dinv = dinv_ref[...]
    pv = p_ref[...]
    a = jnp.maximum((pv[0] + pv[1] + g_ref[...]) * dinv + b_ref[...], 0.0)
    o_ref[...] = jnp.dot(a, w_ref[...],
                         preferred_element_type=jnp.float32) * dinv


def _final_body(p_ref, g_ref, dinv_ref, b_ref, batch_ref, wm1_ref, bm1_ref,
                wm2_ref, bm2_ref, o_ref, sums, cnt):
    i = pl.program_id(0)

    @pl.when(i == 0)
    def _():
        sums[...] = jnp.zeros_like(sums)
        cnt[...] = jnp.zeros_like(cnt)

    dinv = dinv_ref[...]
    pv = p_ref[...]
    a = jnp.maximum((pv[0] + pv[1] + g_ref[...]) * dinv + b_ref[...], 0.0)
    gid = lax.broadcasted_iota(jnp.int32, (_BNP, NGRAPH), 1)
    oh = (batch_ref[...] == gid).astype(jnp.float32)
    sums[...] += lax.dot_general(oh, a, (((0,), (0,)), ((), ())),
                                 preferred_element_type=jnp.float32)
    cnt[...] += jnp.broadcast_to(jnp.sum(oh, axis=0)[:, None], (NGRAPH, D))

    @pl.when(i == pl.num_programs(0) - 1)
    def _():
        pooled = sums[...] / jnp.maximum(cnt[...], 1.0)
        z = jnp.maximum(jnp.dot(pooled, wm1_ref[...],
                                preferred_element_type=jnp.float32)
                        + bm1_ref[...], 0.0)
        o_ref[...] = jnp.dot(z, wm2_ref[...],
                             preferred_element_type=jnp.float32) + bm2_ref[...]


def _mm_scale(x, w, degp):
    return pl.pallas_call(
        _mm_scale_body,
        grid=(N // _BN,),
        in_specs=[pl.BlockSpec((_BN, D), lambda i: (i, 0)),
                  pl.BlockSpec((D, D), lambda i: (0, 0)),
                  pl.BlockSpec((NSC, _BN, 1), lambda i: (0, i, 0))],
        out_specs=[pl.BlockSpec((_BN, D), lambda i: (i, 0)),
                   pl.BlockSpec((_BN, D), lambda i: (i, 0))],
        out_shape=[jax.ShapeDtypeStruct((N, D), jnp.float32),
                   jax.ShapeDtypeStruct((N, D), jnp.float32)],
    )(x, w, degp)


def _layer(p, g, dinvb, b, w):
    return pl.pallas_call(
        _layer_body,
        grid=(N // _BN,),
        in_specs=[pl.BlockSpec((NSC, _BN, D), lambda i: (0, i, 0)),
                  pl.BlockSpec((_BN, D), lambda i: (i, 0)),
                  pl.BlockSpec((_BN, D), lambda i: (i, 0)),
                  pl.BlockSpec((1, D), lambda i: (0, 0)),
                  pl.BlockSpec((D, D), lambda i: (0, 0))],
        out_specs=pl.BlockSpec((_BN, D), lambda i: (i, 0)),
        out_shape=jax.ShapeDtypeStruct((N, D), jnp.float32),
    )(p, g, dinvb, b, w)


def _final(p, g, dinvb, b, batch2, wm1, bm1, wm2, bm2):
    nhid = wm1.shape[1]
    return pl.pallas_call(
        _final_body,
        grid=(N // _BNP,),
        in_specs=[pl.BlockSpec((NSC, _BNP, D), lambda i: (0, i, 0)),
                  pl.BlockSpec((_BNP, D), lambda i: (i, 0)),
                  pl.BlockSpec((_BNP, D), lambda i: (i, 0)),
                  pl.BlockSpec((1, D), lambda i: (0, 0)),
                  pl.BlockSpec((_BNP, 1), lambda i: (i, 0)),
                  pl.BlockSpec((D, nhid), lambda i: (0, 0)),
                  pl.BlockSpec((1, nhid), lambda i: (0, 0)),
                  pl.BlockSpec((nhid, D), lambda i: (0, 0)),
                  pl.BlockSpec((1, D), lambda i: (0, 0))],
        out_specs=pl.BlockSpec((NGRAPH, D), lambda i: (0, 0)),
        out_shape=jax.ShapeDtypeStruct((NGRAPH, D), jnp.float32),
        scratch_shapes=[pltpu.VMEM((NGRAPH, D), jnp.float32),
                        pltpu.VMEM((NGRAPH, D), jnp.float32)],
    )(p, g, dinvb, b, batch2, wm1, bm1, wm2, bm2)


def kernel(x, edge_index, batch, W0, b0, Wh0, bh0, Wh1, bh1, Wh2, bh2,
           Wm1, bm1, Wm2, bm2):
    src_flat = edge_index[0]
    dst3 = edge_index[1].reshape(NW, NCHUNK, CHUNK)
    zerosd = jnp.zeros((RPT, D), jnp.float32)

    degp = _deg_sc(dst3).reshape(NSC, NPAD, 1)
    g, dinvb = _mm_scale(x, W0, degp)

    for b, w in ((b0, Wh0), (bh0, Wh1), (bh1, Wh2)):
        p = _agg_sc(g, src_flat, dst3, zerosd)
        g = _layer(p, g, dinvb, b.reshape(1, D), w)
    p = _agg_sc(g, src_flat, dst3, zerosd)
    return _final(p, g, dinvb, bh2.reshape(1, D), batch.reshape(N, 1),
                  Wm1, bm1.reshape(1, -1), Wm2, bm2.reshape(1, D))


# speedup vs baseline: 1.3456x; 1.0098x over previous
import functools

import jax
import jax.numpy as jnp
from jax import lax
from jax.experimental import pallas as pl
from jax.experimental.pallas import tpu as pltpu
from jax.experimental.pallas import tpu_sc as plsc

N = 10000
E = 320000
D = 128
NGRAPH = 64
NSC = 2
NTILE = 16
NW = NSC * NTILE
EPT = E // NW
CHUNK = 80
NCHUNK = EPT // CHUNK
NPAD = 10240
RPT = NPAD // NTILE

_mesh = plsc.VectorSubcoreMesh(core_axis_name="c", subcore_axis_name="s")


@functools.partial(
    pl.kernel,
    mesh=_mesh,
    out_type=jax.ShapeDtypeStruct((NSC * NPAD,), jnp.float32),
    scratch_types=[
        pltpu.VMEM((NCHUNK, CHUNK), jnp.int32),
        pltpu.VMEM((CHUNK,), jnp.float32),
        pltpu.VMEM((RPT,), jnp.float32),
        pltpu.VMEM_SHARED((NPAD,), jnp.float32),
    ],
)
def _deg_sc(dst_hbm, out_hbm, dbuf, onesbuf, zbuf, acc):
    cid = lax.axis_index("c")
    sid = lax.axis_index("s")
    wid = sid * NSC + cid
    pltpu.sync_copy(dst_hbm.at[wid], dbuf)

    def fill(i, carry):
        onesbuf[pl.ds(i * 16, 16)] = jnp.ones((16,), jnp.float32)
        return carry

    lax.fori_loop(0, CHUNK // 16, fill, 0)

    def zfill(i, carry):
        zbuf[pl.ds(i * 16, 16)] = jnp.zeros((16,), jnp.float32)
        return carry

    lax.fori_loop(0, RPT // 16, zfill, 0)
    pltpu.sync_copy(zbuf, acc.at[pl.ds(sid * RPT, RPT)])
    plsc.subcore_barrier()

    def body(i, carry):
        pltpu.sync_copy(onesbuf, acc.at[dbuf.at[i]], add=True)
        return carry

    lax.fori_loop(0, NCHUNK, body, 0)
    plsc.subcore_barrier()
    pltpu.sync_copy(acc.at[pl.ds(sid * RPT, RPT)],
                    out_hbm.at[pl.ds(cid * NPAD + sid * RPT, RPT)])


@functools.partial(
    pl.kernel,
    mesh=_mesh,
    out_type=jax.ShapeDtypeStruct((NSC, NPAD, D), jnp.float32),
    scratch_types=[
        pltpu.VMEM((EPT,), jnp.int32),
        pltpu.VMEM((NCHUNK, CHUNK), jnp.int32),
        pltpu.VMEM((CHUNK, D), jnp.float32),
        pltpu.VMEM((CHUNK, D), jnp.float32),
        pltpu.VMEM_SHARED((NPAD, D), jnp.float32),
        pltpu.SemaphoreType.DMA,
        pltpu.SemaphoreType.DMA,
    ],
)
def _agg_sc(g_hbm, src_hbm, dst_hbm, zeros_hbm, out_hbm, sbuf, dbuf, rows0,
            rows1, acc, sem0, sem1):
    cid = lax.axis_index("c")
    sid = lax.axis_index("s")
    wid = sid * NSC + cid
    c1 = pltpu.async_copy(src_hbm.at[pl.ds(wid * EPT, EPT)], sbuf, sem0)
    c2 = pltpu.async_copy(dst_hbm.at[wid], dbuf, sem1)
    pltpu.sync_copy(zeros_hbm, acc.at[pl.ds(sid * RPT, RPT)])
    c1.wait()
    c2.wait()
    plsc.subcore_barrier()

    bufs = (rows0, rows1)
    sems = (sem0, sem1)

    def _gather(c, b):
        return pltpu.async_copy(
            g_hbm.at[sbuf.at[pl.ds(c * CHUNK, CHUNK)]], bufs[b], sems[b])

    _gather(0, 0)

    def body(j, carry):
        for b in (0, 1):
            c = 2 * j + b
            _gather(c + 1, 1 - b)
            pltpu.make_async_copy(
                g_hbm.at[sbuf.at[pl.ds(c * CHUNK, CHUNK)]], bufs[b],
                sems[b]).wait()
            pltpu.sync_copy(bufs[b], acc.at[dbuf.at[c]], add=True)
        return carry

    lax.fori_loop(0, NCHUNK // 2, body, 0)
    last = NCHUNK - 1
    pltpu.make_async_copy(
        g_hbm.at[sbuf.at[pl.ds(last * CHUNK, CHUNK)]], bufs[last % 2],
        sems[last % 2]).wait()
    pltpu.sync_copy(bufs[last % 2], acc.at[dbuf.at[last]], add=True)
    plsc.subcore_barrier()
    pltpu.sync_copy(acc.at[pl.ds(sid * RPT, RPT)],
                    out_hbm.at[cid, pl.ds(sid * RPT, RPT)])


_BN = 1000
_BNP = 2000


def _dinv_block(dpv):
    deg = dpv[0, :, 0:1] + dpv[1, :, 0:1] + 1.0
    return lax.rsqrt(deg)


def _mm_scale_body(x_ref, w_ref, dp_ref, o_ref, dinv_ref):
    dinv = _dinv_block(dp_ref[...])
    dinv_ref[...] = jnp.broadcast_to(dinv, dinv_ref.shape)
    o_ref[...] = jnp.dot(x_ref[...], w_ref[...],
                         preferred_element_type=jnp.float32) * dinv


def _layer_body(p_ref, g_ref, dinv_ref, b_ref, w_ref, o_ref):
    dinv = dinv_ref[...]
    pv = p_ref[...]
    a = jnp.maximum((pv[0] + pv[1] + g_ref[...]) * dinv + b_ref[...], 0.0)
    o_ref[...] = jnp.dot(a, w_ref[...],
                         preferred_element_type=jnp.float32) * dinv


def _final_body(p_ref, g_ref, dinv_ref, b_ref, batch_ref, wm1_ref, bm1_ref,
                wm2_ref, bm2_ref, o_ref, sums, cnt):
    i = pl.program_id(0)

    @pl.when(i == 0)
    def _():
        sums[...] = jnp.zeros_like(sums)
        cnt[...] = jnp.zeros_like(cnt)

    dinv = dinv_ref[...]
    pv = p_ref[...]
    a = jnp.maximum((pv[0] + pv[1] + g_ref[...]) * dinv + b_ref[...], 0.0)
    gid = lax.broadcasted_iota(jnp.int32, (_BNP, NGRAPH), 1)
    oh = (batch_ref[...] == gid).astype(jnp.float32)
    sums[...] += lax.dot_general(oh, a, (((0,), (0,)), ((), ())),
                                 preferred_element_type=jnp.float32)
    cnt[...] += jnp.broadcast_to(jnp.sum(oh, axis=0)[:, None], (NGRAPH, D))

    @pl.when(i == pl.num_programs(0) - 1)
    def _():
        pooled = sums[...] / jnp.maximum(cnt[...], 1.0)
        z = jnp.maximum(jnp.dot(pooled, wm1_ref[...],
                                preferred_element_type=jnp.float32)
                        + bm1_ref[...], 0.0)
        o_ref[...] = jnp.dot(z, wm2_ref[...],
                             preferred_element_type=jnp.float32) + bm2_ref[...]


def _mm_scale(x, w, degp):
    return pl.pallas_call(
        _mm_scale_body,
        grid=(N // _BN,),
        in_specs=[pl.BlockSpec((_BN, D), lambda i: (i, 0)),
                  pl.BlockSpec((D, D), lambda i: (0, 0)),
                  pl.BlockSpec((NSC, _BN, 1), lambda i: (0, i, 0))],
        out_specs=[pl.BlockSpec((_BN, D), lambda i: (i, 0)),
                   pl.BlockSpec((_BN, D), lambda i: (i, 0))],
        out_shape=[jax.ShapeDtypeStruct((N, D), jnp.float32),
                   jax.ShapeDtypeStruct((N, D), jnp.float32)],
    )(x, w, degp)


def _layer(p, g, dinvb, b, w):
    return pl.pallas_call(
        _layer_body,
        grid=(N // _BN,),
        in_specs=[pl.BlockSpec((NSC, _BN, D), lambda i: (0, i, 0)),
                  pl.BlockSpec((_BN, D), lambda i: (i, 0)),
                  pl.BlockSpec((_BN, D), lambda i: (i, 0)),
                  pl.BlockSpec((1, D), lambda i: (0, 0)),
                  pl.BlockSpec((D, D), lambda i: (0, 0))],
        out_specs=pl.BlockSpec((_BN, D), lambda i: (i, 0)),
        out_shape=jax.ShapeDtypeStruct((N, D), jnp.float32),
    )(p, g, dinvb, b, w)


def _final(p, g, dinvb, b, batch2, wm1, bm1, wm2, bm2):
    nhid = wm1.shape[1]
    return pl.pallas_call(
        _final_body,
        grid=(N // _BNP,),
        in_specs=[pl.BlockSpec((NSC, _BNP, D), lambda i: (0, i, 0)),
                  pl.BlockSpec((_BNP, D), lambda i: (i, 0)),
                  pl.BlockSpec((_BNP, D), lambda i: (i, 0)),
                  pl.BlockSpec((1, D), lambda i: (0, 0)),
                  pl.BlockSpec((_BNP, 1), lambda i: (i, 0)),
                  pl.BlockSpec((D, nhid), lambda i: (0, 0)),
                  pl.BlockSpec((1, nhid), lambda i: (0, 0)),
                  pl.BlockSpec((nhid, D), lambda i: (0, 0)),
                  pl.BlockSpec((1, D), lambda i: (0, 0))],
        out_specs=pl.BlockSpec((NGRAPH, D), lambda i: (0, 0)),
        out_shape=jax.ShapeDtypeStruct((NGRAPH, D), jnp.float32),
        scratch_shapes=[pltpu.VMEM((NGRAPH, D), jnp.float32),
                        pltpu.VMEM((NGRAPH, D), jnp.float32)],
    )(p, g, dinvb, b, batch2, wm1, bm1, wm2, bm2)


def kernel(x, edge_index, batch, W0, b0, Wh0, bh0, Wh1, bh1, Wh2, bh2,
           Wm1, bm1, Wm2, bm2):
    src_flat = edge_index[0]
    dst3 = edge_index[1].reshape(NW, NCHUNK, CHUNK)
    zerosd = jnp.zeros((RPT, D), jnp.float32)

    degp = _deg_sc(dst3).reshape(NSC, NPAD, 1)
    g, dinvb = _mm_scale(x, W0, degp)

    for b, w in ((b0, Wh0), (bh0, Wh1), (bh1, Wh2)):
        p = _agg_sc(g, src_flat, dst3, zerosd)
        g = _layer(p, g, dinvb, b.reshape(1, D), w)
    p = _agg_sc(g, src_flat, dst3, zerosd)
    return _final(p, g, dinvb, bh2.reshape(1, D), batch.reshape(N, 1),
                  Wm1, bm1.reshape(1, -1), Wm2, bm2.reshape(1, D))
